# baseline (speedup 1.0000x reference)
import jax
import jax.numpy as jnp
from jax import lax
from jax.experimental import pallas as pl
from jax.experimental.pallas import tpu as pltpu
from jax.experimental.pallas import tpu_sc as plsc
import functools

N = 8192
ROWS = 128
NUM_CORES = 2
NUM_SUBCORES = 16
NWORK = NUM_CORES * NUM_SUBCORES
RPW = ROWS // NWORK
E = 8
ESIZE = N // E
EV = ESIZE // 16
R = 2048
SHIFTS = (0, 11, 22)
MASKS = (2047, 2047, 1023)
MU = (N - 1) / 2.0
DEN = float(N) * (float(N) * float(N) - 1.0) / 12.0
MIN_I32 = -(2**31)


def _sc_body(pred_hbm, targ_hbm, out_hbm,
             kp, kt, pp0, pp1, pt0, pt1, rank_p,
             hist_p, cnt_p, hist_t, cnt_t, lossbuf):
    lane = lax.iota(jnp.int32, 16)

    def digit(key, p):
        return jnp.bitwise_and(jnp.right_shift(key, SHIFTS[p]), MASKS[p])

    def zero_hists():
        def body(c, _):
            hist_p[pl.ds(c * 16, 16)] = jnp.zeros(16, jnp.int32)
            hist_t[pl.ds(c * 16, 16)] = jnp.zeros(16, jnp.int32)
            return 0
        lax.fori_loop(0, (E * R) // 16, body, 0)

    def sweep0():
        @plsc.parallel_loop(0, ESIZE, 16, unroll=2)
        def _(off):
            for kref, hist in ((kp, hist_p), (kt, hist_t)):
                for e in range(E):
                    o = e * ESIZE + off
                    u = kref[pl.ds(o, 16)]
                    m = jnp.right_shift(u, 31)
                    key = jnp.bitwise_xor(u, jnp.bitwise_or(m, MIN_I32))
                    kref[pl.ds(o, 16)] = key
                    d = digit(key, 0)
                    counts, last = plsc.scan_count(d)
                    plsc.addupdate_scatter(
                        hist.at[pl.ds(e * R, R)], [d], counts, mask=last)

    def prefix_dual(nbins):
        def body(c, carry):
            cp, ct = carry
            off = c * 16
            zeros = jnp.zeros(16, jnp.int32)

            def one(hist, cnt, carry_s):
                hs = [hist[pl.ds(e * R + off, 16)] for e in range(E)]
                tot = hs[0]
                for e in range(1, E):
                    tot = tot + hs[e]
                base = (plsc.cumsum(tot) - tot) + carry_s
                for e in range(E):
                    cnt[pl.ds(e * R + off, 16)] = base
                    hist[pl.ds(e * R + off, 16)] = zeros
                    if e + 1 < E:
                        base = base + hs[e]
                return carry_s + jnp.sum(tot)

            return one(hist_p, cnt_p, cp), one(hist_t, cnt_t, ct)
        lax.fori_loop(0, nbins // 16, body, (jnp.int32(0), jnp.int32(0)))

    def permute(p, srcs):
        def body(c, _):
            front = []
            for (kref, pa, pb, hist, cnt) in srcs:
                for e in range(E):
                    off = e * ESIZE + c * 16
                    if pa is None:
                        pay = off + lane
                        key = kref[pl.ds(off, 16)]
                    else:
                        pay = pa[pl.ds(off, 16)]
                        key = plsc.load_gather(kref, [pay])
                    d = digit(key, p)
                    counts, last = plsc.scan_count(d)
                    front.append((e, pay, key, d, counts, last,
                                  pb, hist, cnt))
            poss = []
            for (e, pay, key, d, counts, last, pb, hist, cnt) in front:
                cs = cnt.at[pl.ds(e * R, R)]
                base = plsc.load_gather(cs, [d])
                pos = base + counts - 1
                plsc.store_scatter(cs, [d], pos + 1, mask=last)
                poss.append(pos)
            for pos, (e, pay, key, d, counts, last, pb, hist, cnt) in zip(
                    poss, front):
                plsc.store_scatter(pb, [pos], pay)
                dn = digit(key, p + 1)
                idx = jnp.right_shift(pos, 10) * R + dn
                cn, ln = plsc.scan_count(idx)
                plsc.addupdate_scatter(hist, [idx], cn, mask=ln)
            return 0
        lax.fori_loop(0, EV, body, 0)

    def final_pred():
        def body(c, _):
            front = []
            for e in range(E):
                off = e * ESIZE + c * 16
                pay = pp0[pl.ds(off, 16)]
                key = plsc.load_gather(kp, [pay])
                d = digit(key, 2)
                counts, last = plsc.scan_count(d)
                front.append((e, pay, d, counts, last))
            work = []
            for (e, pay, d, counts, last) in front:
                cs = cnt_p.at[pl.ds(e * R, R)]
                base = plsc.load_gather(cs, [d])
                pos = base + counts - 1
                plsc.store_scatter(cs, [d], pos + 1, mask=last)
                work.append((pay, pos))
            for pay, pos in work:
                plsc.store_scatter(rank_p, [pay], pos)
            return 0
        lax.fori_loop(0, EV, body, 0)

    def final_targ():
        def body(c, acc):
            front = []
            for e in range(E):
                off = e * ESIZE + c * 16
                pay = pt0[pl.ds(off, 16)]
                key = plsc.load_gather(kt, [pay])
                d = digit(key, 2)
                counts, last = plsc.scan_count(d)
                front.append((e, pay, d, counts, last))
            prods = []
            for (e, pay, d, counts, last) in front:
                cs = cnt_t.at[pl.ds(e * R, R)]
                base = plsc.load_gather(cs, [d])
                pos = base + counts - 1
                plsc.store_scatter(cs, [d], pos + 1, mask=last)
                rp = plsc.load_gather(rank_p, [pay])
                prods.append((pos, rp))
            for pos, rp in prods:
                acc = acc + (pos.astype(jnp.float32) - MU) * (
                    rp.astype(jnp.float32) - MU)
            return acc
        return lax.fori_loop(0, EV, body, jnp.zeros(16, jnp.float32))

    wid = lax.axis_index("s") * NUM_CORES + lax.axis_index("c")
    zero_hists()

    def row_body(j, loss_vec):
        row = wid * RPW + j
        pltpu.sync_copy(pred_hbm.at[row], kp)
        pltpu.sync_copy(targ_hbm.at[row], kt)
        sweep0()
        prefix_dual(R)
        permute(0, ((kp, None, pp1, hist_p, cnt_p),
                    (kt, None, pt1, hist_t, cnt_t)))
        prefix_dual(R)
        permute(1, ((kp, pp1, pp0, hist_p, cnt_p),
                    (kt, pt1, pt0, hist_t, cnt_t)))
        prefix_dual(1024)
        final_pred()
        acc = final_targ()
        s = jnp.sum(acc)
        return jnp.where(lane == j, 1.0 - s * (1.0 / DEN), loss_vec)

    loss_vec = lax.fori_loop(0, RPW, row_body, jnp.zeros(16, jnp.float32))
    lossbuf[...] = loss_vec
    pltpu.sync_copy(lossbuf, out_hbm.at[wid])


@jax.jit
def kernel(predictions, targets):
    mesh = plsc.VectorSubcoreMesh(
        core_axis_name="c", subcore_axis_name="s",
        num_cores=NUM_CORES, num_subcores=NUM_SUBCORES)
    run = functools.partial(
        pl.kernel,
        out_type=jax.ShapeDtypeStruct((NWORK, 16), jnp.float32),
        mesh=mesh,
        compiler_params=pltpu.CompilerParams(needs_layout_passes=False),
        scratch_types=[
            pltpu.VMEM((N,), jnp.int32),
            pltpu.VMEM((N,), jnp.int32),
            pltpu.VMEM((N,), jnp.int32),
            pltpu.VMEM((N,), jnp.int32),
            pltpu.VMEM((N,), jnp.int32),
            pltpu.VMEM((N,), jnp.int32),
            pltpu.VMEM((N,), jnp.int32),
            pltpu.VMEM((E * R,), jnp.int32),
            pltpu.VMEM((E * R,), jnp.int32),
            pltpu.VMEM((E * R,), jnp.int32),
            pltpu.VMEM((E * R,), jnp.int32),
            pltpu.VMEM((16,), jnp.float32),
        ],
    )(_sc_body)
    pred_bits = lax.bitcast_convert_type(predictions, jnp.int32)
    targ_bits = lax.bitcast_convert_type(targets, jnp.int32)
    out = run(pred_bits, targ_bits)
    return jnp.sum(out) * (1.0 / ROWS)
